# baseline (speedup 1.0000x reference)
import functools

import jax
import jax.numpy as jnp
from jax import lax
from jax.experimental import pallas as pl
from jax.experimental.pallas import tpu as pltpu
from jax.experimental.pallas import tpu_sc as plsc

BATCH = 16384
DIM = 32
_TC_N = 3584
_SC_N = BATCH - _TC_N

_info = plsc.get_sparse_core_info()
_NC, _NS, _L = _info.num_cores, _info.num_subcores, _info.num_lanes
_NW = _NC * _NS
_BPW = _SC_N // _NW
_CHUNKS = _BPW // _L
_K = 8
_LOOK = 7

_TK = 8
_TLOOK = 6


def _sc_body(user_ids_hbm, item_ids_hbm, ut_hbm, it_hbm, out_hbm,
             uidx_v, iidx_v, tu_v, wu_v, ti_v, wi_v,
             u_ring, i_ring, out_v, sem_u, sem_i):
    wid = lax.axis_index("s") * _NC + lax.axis_index("c")
    base = wid * _BPW

    pltpu.sync_copy(user_ids_hbm.at[pl.ds(base, _BPW)], uidx_v)
    pltpu.sync_copy(item_ids_hbm.at[pl.ds(base, _BPW)], iidx_v)

    def pre_body(c, carry):
        sl = pl.ds(c * _L, _L)
        ju = uidx_v[sl]
        ji = iidx_v[sl]
        tu_v[sl] = lax.shift_right_logical(ju, 7)
        wu_v[sl] = lax.bitwise_and(ju, 127)
        ti_v[sl] = lax.shift_right_logical(ji, 7)
        wi_v[sl] = lax.bitwise_and(ji, 127)
        return carry

    lax.fori_loop(0, _CHUNKS, pre_body, 0)

    lane = lax.iota(jnp.int32, _L)
    lane_hi = lane + _L

    def fire(tu, ti, slot):
        pltpu.async_copy(
            ut_hbm.at[:, pl.ds(pl.multiple_of(tu * 128, 128), 128)],
            u_ring.at[:, pl.ds(slot * 128, 128)], sem_u.at[slot])
        pltpu.async_copy(
            it_hbm.at[:, pl.ds(pl.multiple_of(ti * 128, 128), 128)],
            i_ring.at[:, pl.ds(slot * 128, 128)], sem_i.at[slot])

    def drain(slot):
        pltpu.make_async_copy(
            ut_hbm.at[:, pl.ds(0, 128)],
            u_ring.at[:, pl.ds(slot * 128, 128)], sem_u.at[slot]).wait()
        pltpu.make_async_copy(
            it_hbm.at[:, pl.ds(0, 128)],
            i_ring.at[:, pl.ds(slot * 128, 128)], sem_i.at[slot]).wait()

    def extract(wu, wi, slot):
        cu = jnp.full((_L,), slot * 128 + wu, jnp.int32)
        ci = jnp.full((_L,), slot * 128 + wi, jnp.int32)
        u_lo = plsc.load_gather(u_ring, [lane, cu])
        u_hi = plsc.load_gather(u_ring, [lane_hi, cu])
        i_lo = plsc.load_gather(i_ring, [lane, ci])
        i_hi = plsc.load_gather(i_ring, [lane_hi, ci])
        return jnp.sum(u_lo * i_lo + u_hi * i_hi)

    tu0 = tu_v[pl.ds(0, _L)]
    ti0 = ti_v[pl.ds(0, _L)]
    for l in range(_LOOK):
        fire(tu0[l], ti0[l], l % _K)

    def chunk_body(c, carry):
        sl = pl.ds(c * _L, _L)
        wu_cur = wu_v[sl]
        wi_cur = wi_v[sl]
        tu_cur = tu_v[sl]
        ti_cur = ti_v[sl]
        nxt = pl.ds(jnp.minimum(c + 1, _CHUNKS - 1) * _L, _L)
        tu_nxt = tu_v[nxt]
        ti_nxt = ti_v[nxt]
        acc = jnp.zeros((_L,), jnp.float32)
        for l in range(_L):
            lf = l + _LOOK
            if lf < _L:
                fire(tu_cur[lf], ti_cur[lf], lf % _K)
            else:
                tun = tu_nxt[lf - _L]
                tin = ti_nxt[lf - _L]
                pl.when(c < _CHUNKS - 1)(
                    lambda tun=tun, tin=tin, lf=lf: fire(tun, tin, lf % _K))
            slot = l % _K
            drain(slot)
            s = extract(wu_cur[l], wi_cur[l], slot)
            acc = jnp.where(lane == l, s, acc)
        out_v[sl] = acc
        return carry

    lax.fori_loop(0, _CHUNKS, chunk_body, 0)

    pltpu.sync_copy(out_v, out_hbm.at[pl.ds(base, _BPW)])


def _tc_body(uids_ref, iids_ref, ut_ref, it_ref, out_ref,
             uwin, iwin, sem_u, sem_i):
    lane = lax.broadcasted_iota(jnp.int32, (1, 128), 1)
    col = lax.broadcasted_iota(jnp.int32, (DIM, 128), 1)

    def fire(q):
        slot = lax.rem(q, _TK)
        ju = uids_ref[q]
        ji = iids_ref[q]
        pltpu.make_async_copy(
            ut_ref.at[:, pl.ds(pl.multiple_of(
                lax.shift_right_logical(ju, 7) * 128, 128), 128)],
            uwin.at[slot], sem_u.at[slot]).start()
        pltpu.make_async_copy(
            it_ref.at[:, pl.ds(pl.multiple_of(
                lax.shift_right_logical(ji, 7) * 128, 128), 128)],
            iwin.at[slot], sem_i.at[slot]).start()

    def body(q, acc):
        pl.when(q < _TC_N)(lambda: fire(q))

        p = q - _TLOOK

        def do_extract():
            slot = lax.rem(p, _TK)
            pltpu.make_async_copy(
                ut_ref.at[:, pl.ds(0, 128)], uwin.at[slot],
                sem_u.at[slot]).wait()
            pltpu.make_async_copy(
                it_ref.at[:, pl.ds(0, 128)], iwin.at[slot],
                sem_i.at[slot]).wait()
            wu = lax.bitwise_and(uids_ref[p], 127)
            wi = lax.bitwise_and(iids_ref[p], 127)
            u = uwin[slot]
            i = iwin[slot]
            us = jnp.sum(jnp.where(col == wu, u, 0.0), axis=1, keepdims=True)
            is_ = jnp.sum(jnp.where(col == wi, i, 0.0), axis=1, keepdims=True)
            s = jnp.sum(us * is_)
            acc2 = jnp.where(lane == lax.rem(p, 128), s, acc)

            def flush():
                out_ref[pl.ds(lax.div(p, 128), 1), :] = acc2

            pl.when(lax.rem(p, 128) == 127)(flush)
            return jnp.where(lax.rem(p, 128) == 127,
                             jnp.zeros_like(acc2), acc2)

        return lax.cond(p >= 0, do_extract, lambda: acc)

    lax.fori_loop(0, _TC_N + _TLOOK, body, jnp.zeros((1, 128), jnp.float32))


@jax.jit
def _two_tower(user_ids, item_ids, user_emb, item_emb):
    ut = user_emb.T
    it = item_emb.T

    mesh = plsc.VectorSubcoreMesh(core_axis_name="c", subcore_axis_name="s")
    sc_kern = pl.kernel(
        _sc_body,
        mesh=mesh,
        compiler_params=pltpu.CompilerParams(
            needs_layout_passes=False, use_tc_tiling_on_sc=True),
        out_type=jax.ShapeDtypeStruct((_SC_N,), jnp.float32),
        scratch_types=[
            pltpu.VMEM((_BPW,), jnp.int32),
            pltpu.VMEM((_BPW,), jnp.int32),
            pltpu.VMEM((_BPW,), jnp.int32),
            pltpu.VMEM((_BPW,), jnp.int32),
            pltpu.VMEM((_BPW,), jnp.int32),
            pltpu.VMEM((_BPW,), jnp.int32),
            pltpu.VMEM((DIM, _K * 128), jnp.float32),
            pltpu.VMEM((DIM, _K * 128), jnp.float32),
            pltpu.VMEM((_BPW,), jnp.float32),
            pltpu.SemaphoreType.DMA((_K,)),
            pltpu.SemaphoreType.DMA((_K,)),
        ],
    )
    sc_out = sc_kern(user_ids[:_SC_N], item_ids[:_SC_N], ut, it)

    tc_out = pl.pallas_call(
        _tc_body,
        out_shape=jax.ShapeDtypeStruct((_TC_N // 128, 128), jnp.float32),
        in_specs=[
            pl.BlockSpec(memory_space=pltpu.SMEM),
            pl.BlockSpec(memory_space=pltpu.SMEM),
            pl.BlockSpec(memory_space=pltpu.HBM),
            pl.BlockSpec(memory_space=pltpu.HBM),
        ],
        scratch_shapes=[
            pltpu.VMEM((_TK, DIM, 128), jnp.float32),
            pltpu.VMEM((_TK, DIM, 128), jnp.float32),
            pltpu.SemaphoreType.DMA((_TK,)),
            pltpu.SemaphoreType.DMA((_TK,)),
        ],
    )(user_ids[_SC_N:], item_ids[_SC_N:], ut, it)

    return jnp.concatenate([sc_out, tc_out.reshape(_TC_N)])


def kernel(user_ids, item_ids, user_emb, item_emb):
    return _two_tower(user_ids.astype(jnp.int32), item_ids.astype(jnp.int32),
                      user_emb, item_emb)
